# Initial kernel scaffold; baseline (speedup 1.0000x reference)
import jax
import jax.numpy as jnp
from jax.experimental import pallas as pl


def kernel(x, edge_index, W1, b1, beta2):
    raise NotImplementedError("write your pallas kernel here")


# speedup vs baseline: 10.3072x; 10.3072x over previous
import functools

import jax
import jax.numpy as jnp
from jax import lax
from jax.experimental import pallas as pl
from jax.experimental.pallas import tpu as pltpu
from jax.experimental.pallas import tpu_sc as plsc

NC = 2
NS = 16
NW = NC * NS
CH = 80


def _lin_body(x_ref, w_ref, b_ref, h_ref, hn_ref):
    h = lax.dot_general(x_ref[...], w_ref[...],
                        (((1,), (1,)), ((), ())),
                        preferred_element_type=jnp.float32) + b_ref[...]
    h_ref[...] = h
    norm = jnp.sqrt(jnp.sum(h * h, axis=1, keepdims=True))
    hn_ref[...] = h / jnp.maximum(norm, 1e-12)


def _linear_norm(x, W1, b1):
    n, d = x.shape
    br = 1000
    return pl.pallas_call(
        _lin_body,
        grid=(n // br,),
        in_specs=[pl.BlockSpec((br, d), lambda i: (i, 0)),
                  pl.BlockSpec((d, d), lambda i: (0, 0)),
                  pl.BlockSpec((1, d), lambda i: (0, 0))],
        out_specs=[pl.BlockSpec((br, d), lambda i: (i, 0)),
                   pl.BlockSpec((br, d), lambda i: (i, 0))],
        out_shape=[jax.ShapeDtypeStruct((n, d), jnp.float32)] * 2,
    )(x, W1, b1.reshape(1, d))


def _merge_norm_body(num_ref, den0_ref, den1_ref, h_ref, hn_ref):
    numer = num_ref[0] + num_ref[1]
    den = den0_ref[...] + den1_ref[...]
    out = numer / (den + 1e-16)
    h_ref[...] = out
    norm = jnp.sqrt(jnp.sum(out * out, axis=1, keepdims=True))
    hn_ref[...] = out / jnp.maximum(norm, 1e-12)


def _merge_norm(num, den0, den1, n):
    d = num.shape[2]
    br = 1000
    return pl.pallas_call(
        _merge_norm_body,
        grid=(n // br,),
        in_specs=[pl.BlockSpec((NC, br, d), lambda i: (0, i, 0)),
                  pl.BlockSpec((br, 1), lambda i: (i, 0)),
                  pl.BlockSpec((br, 1), lambda i: (i, 0))],
        out_specs=[pl.BlockSpec((br, d), lambda i: (i, 0)),
                   pl.BlockSpec((br, d), lambda i: (i, 0))],
        out_shape=[jax.ShapeDtypeStruct((n, d), jnp.float32)] * 2,
    )(num, den0, den1)


def _merge_lsm_body(num_ref, den0_ref, den1_ref, o_ref):
    numer = num_ref[0] + num_ref[1]
    den = den0_ref[...] + den1_ref[...]
    h = numer / (den + 1e-16)
    m = jnp.max(h, axis=1, keepdims=True)
    ex = jnp.exp(h - m)
    lse = jnp.log(jnp.sum(ex, axis=1, keepdims=True))
    o_ref[...] = h - m - lse


def _merge_lsm(num, den0, den1, n):
    d = num.shape[2]
    br = 1000
    return pl.pallas_call(
        _merge_lsm_body,
        grid=(n // br,),
        in_specs=[pl.BlockSpec((NC, br, d), lambda i: (0, i, 0)),
                  pl.BlockSpec((br, 1), lambda i: (i, 0)),
                  pl.BlockSpec((br, 1), lambda i: (i, 0))],
        out_specs=pl.BlockSpec((br, d), lambda i: (i, 0)),
        out_shape=jax.ShapeDtypeStruct((n, d), jnp.float32),
    )(num, den0, den1)


def _sc_conv(h, hn, src, dst, beta16, n_pad):
    n, d = h.shape
    e = src.shape[0]
    per_w = e // NW
    n_chunks = per_w // CH
    rows_t = n_pad // NS
    mesh = plsc.VectorSubcoreMesh(core_axis_name="c", subcore_axis_name="s",
                                  num_cores=NC, num_subcores=NS)

    @functools.partial(
        pl.kernel,
        out_type=(jax.ShapeDtypeStruct((NC, n_pad, d), jnp.float32),
                  jax.ShapeDtypeStruct((n_pad,), jnp.float32),
                  jax.ShapeDtypeStruct((n_pad,), jnp.float32)),
        mesh=mesh,
        compiler_params=pltpu.CompilerParams(needs_layout_passes=False),
        scratch_types=[
            pltpu.VMEM((CH,), jnp.int32),
            pltpu.VMEM((CH,), jnp.int32),
            pltpu.VMEM((CH, d), jnp.float32),
            pltpu.VMEM((CH, d), jnp.float32),
            pltpu.VMEM((CH, d), jnp.float32),
            pltpu.VMEM((CH * 16,), jnp.float32),
            pltpu.VMEM((CH,), jnp.float32),
            pltpu.VMEM((n_pad // NS,), jnp.float32),
            pltpu.VMEM((16,), jnp.float32),
            pltpu.VMEM_SHARED((n_pad, d), jnp.float32),
            pltpu.VMEM_SHARED((n_pad,), jnp.float32),
            pltpu.SemaphoreType.DMA,
            pltpu.SemaphoreType.DMA,
            pltpu.SemaphoreType.DMA,
        ],
    )
    def conv(h_hbm, hn_hbm, src_hbm, dst_hbm, beta_hbm,
             num_hbm, den0_hbm, den1_hbm,
             src_v, dst_v, a_rows, b_rows, h_rows, pb, ew, dzb, beta_v,
             num_sh, den_sh, sem1, sem2, sem3):
        cid = lax.axis_index("c")
        sid = lax.axis_index("s")
        wid = sid * NC + cid
        lanes = lax.iota(jnp.int32, 16)
        zeros16f = jnp.zeros((16,), jnp.float32)

        pltpu.sync_copy(beta_hbm, beta_v)

        @pl.loop(0, CH)
        def _z0(i):
            for k in range(d // 16):
                a_rows[i, pl.ds(16 * k, 16)] = zeros16f

        @pl.loop(0, rows_t // 16)
        def _z0b(i):
            dzb[pl.ds(i * 16, 16)] = zeros16f

        pltpu.sync_copy(dzb, den_sh.at[pl.ds(sid * rows_t, rows_t)])

        @pl.loop(0, rows_t // CH)
        def _z1(i):
            base = sid * rows_t + i * CH
            pltpu.sync_copy(a_rows, num_sh.at[pl.ds(base, CH)])

        plsc.subcore_barrier()

        @pl.loop(0, n_chunks)
        def _chunk(ci):
            base = wid * per_w + ci * CH
            pltpu.sync_copy(src_hbm.at[pl.ds(base, CH)], src_v)
            pltpu.sync_copy(dst_hbm.at[pl.ds(base, CH)], dst_v)
            cp1 = pltpu.async_copy(hn_hbm.at[dst_v], a_rows, sem1)
            cp2 = pltpu.async_copy(hn_hbm.at[src_v], b_rows, sem2)
            cp3 = pltpu.async_copy(h_hbm.at[src_v], h_rows, sem3)
            cp1.wait()
            cp2.wait()

            @pl.loop(0, CH)
            def _dot(i):
                acc = a_rows[i, pl.ds(0, 16)] * b_rows[i, pl.ds(0, 16)]
                for k in range(1, d // 16):
                    acc = acc + (a_rows[i, pl.ds(16 * k, 16)]
                                 * b_rows[i, pl.ds(16 * k, 16)])
                pb[pl.ds(i * 16, 16)] = acc

            cp3.wait()

            @pl.loop(0, CH // 16)
            def _red(g):
                rowbase = (g * 16 + lanes) * 16
                t = plsc.load_gather(pb, [rowbase])
                for c in range(1, 16):
                    t = t + plsc.load_gather(pb, [rowbase + c])
                e16 = jnp.exp(t * beta_v[...])
                ew[pl.ds(g * 16, 16)] = e16
                for j in range(16):
                    w = e16[j]
                    row = g * 16 + j
                    for k in range(d // 16):
                        h_rows[row, pl.ds(16 * k, 16)] = (
                            h_rows[row, pl.ds(16 * k, 16)] * w)

            pltpu.sync_copy(h_rows, num_sh.at[dst_v], add=True)
            pltpu.sync_copy(ew, den_sh.at[dst_v], add=True)

        plsc.subcore_barrier()

        @pl.loop(0, rows_t // CH)
        def _out(i):
            base = sid * rows_t + i * CH
            pltpu.sync_copy(num_sh.at[pl.ds(base, CH)], a_rows)
            pltpu.sync_copy(a_rows, num_hbm.at[cid, pl.ds(base, CH)])

        pltpu.sync_copy(den_sh.at[pl.ds(sid * rows_t, rows_t)], dzb)

        @pl.when(cid == 0)
        def _d0():
            pltpu.sync_copy(dzb, den0_hbm.at[pl.ds(sid * rows_t, rows_t)])

        @pl.when(cid == 1)
        def _d1():
            pltpu.sync_copy(dzb, den1_hbm.at[pl.ds(sid * rows_t, rows_t)])

    return conv(h, hn, src, dst, beta16)


def kernel(x, edge_index, W1, b1, beta2):
    n, d = x.shape
    n_pad = ((n + NS * CH - 1) // (NS * CH)) * (NS * CH)
    src = edge_index[0]
    dst = edge_index[1]

    h, hn = _linear_norm(x, W1, b1)
    one16 = jnp.ones((16,), jnp.float32)
    beta16 = jnp.broadcast_to(beta2.astype(jnp.float32), (16,))

    num1, den1a, den1b = _sc_conv(h, hn, src, dst, one16, n_pad)
    h1, hn1 = _merge_norm(num1, den1a.reshape(n_pad, 1),
                          den1b.reshape(n_pad, 1), n)
    num2, den2a, den2b = _sc_conv(h1, hn1, src, dst, beta16, n_pad)
    return _merge_lsm(num2, den2a.reshape(n_pad, 1),
                      den2b.reshape(n_pad, 1), n)
